# baseline (speedup 1.0000x reference)
import functools

import jax
import jax.numpy as jnp
from jax.experimental import pallas as pl
from jax.experimental.pallas import tpu as pltpu

NUM_CB = 4
K = 1024
D = 256
N = 16384
BT = 1024
GRID = N // BT


def _rvq_block_kernel(z_ref, cb_ref, zq_ref, codes_ref, loss_ref):
    z = z_ref[...]
    r = z
    zq = jnp.zeros_like(z)
    loss_sum = jnp.float32(0.0)
    codes = []
    for s in range(NUM_CB):
        cb = cb_ref[s]
        rnorm = jnp.sum(r * r, axis=1, keepdims=True)
        cnorm = jnp.sum(cb * cb, axis=1)
        dots = jax.lax.dot_general(
            r.astype(jnp.bfloat16), cb.astype(jnp.bfloat16),
            (((1,), (1,)), ((), ())),
            preferred_element_type=jnp.float32)
        dists = rnorm + cnorm[None, :] - 2.0 * dots
        idx = jnp.argmin(dists, axis=1).astype(jnp.int32)
        codes.append(idx)
        onehot = (jax.lax.broadcasted_iota(jnp.int32, (BT, K), 1)
                  == idx[:, None]).astype(jnp.float32)
        zqi = jax.lax.dot_general(
            onehot, cb, (((1,), (0,)), ((), ())),
            preferred_element_type=jnp.float32,
            precision=jax.lax.Precision.HIGHEST)
        r = r - zqi
        zq = zq + zqi
        loss_sum = loss_sum + jnp.sum(r * r)
    zq_ref[...] = z + (zq - z)
    codes_ref[...] = jnp.stack(codes, axis=1)
    lane = jax.lax.broadcasted_iota(jnp.int32, (1, 1, 8), 2)
    loss_ref[...] = jnp.where(lane == 0, loss_sum, 0.0)


@jax.jit
def kernel(z, codebooks):
    zq, codes, loss_parts = pl.pallas_call(
        _rvq_block_kernel,
        grid=(GRID,),
        in_specs=[
            pl.BlockSpec((BT, D), lambda i: (i, 0)),
            pl.BlockSpec((NUM_CB, K, D), lambda i: (0, 0, 0)),
        ],
        out_specs=[
            pl.BlockSpec((BT, D), lambda i: (i, 0)),
            pl.BlockSpec((BT, NUM_CB), lambda i: (i, 0)),
            pl.BlockSpec((1, 1, 8), lambda i: (i, 0, 0)),
        ],
        out_shape=[
            jax.ShapeDtypeStruct((N, D), jnp.float32),
            jax.ShapeDtypeStruct((N, NUM_CB), jnp.int32),
            jax.ShapeDtypeStruct((GRID, 1, 8), jnp.float32),
        ],
    )(z, codebooks)
    loss = jnp.sum(loss_parts) / jnp.float32(N * D)
    return zq, codes, loss, loss


# speedup vs baseline: 1.0688x; 1.0688x over previous
import jax
import jax.numpy as jnp
from jax.experimental import pallas as pl
from jax.experimental.pallas import tpu as pltpu

NUM_CB = 4
K = 1024
D = 256
N = 16384
BT = 1024
GRID = N // BT


def _rvq_block_kernel(z_ref, cb_ref, zq_ref, codes_ref, loss_ref, cnorm_ref):
    @pl.when(pl.program_id(0) == 0)
    def _init_cnorm():
        cb_all = cb_ref[...]
        cnorm_ref[...] = jnp.sum(cb_all * cb_all, axis=2)

    z = z_ref[...]
    r = z
    zq = jnp.zeros_like(z)
    loss_sum = jnp.float32(0.0)
    codes = []
    for s in range(NUM_CB):
        cb = cb_ref[s]
        dots = jax.lax.dot_general(
            r.astype(jnp.bfloat16), cb.astype(jnp.bfloat16),
            (((1,), (1,)), ((), ())),
            preferred_element_type=jnp.float32)
        dists = cnorm_ref[s][None, :] - 2.0 * dots
        idx = jnp.argmin(dists, axis=1).astype(jnp.int32)
        codes.append(idx)
        onehot = (jax.lax.broadcasted_iota(jnp.int32, (BT, K), 1)
                  == idx[:, None]).astype(jnp.float32)
        zqi = jax.lax.dot_general(
            onehot, cb, (((1,), (0,)), ((), ())),
            preferred_element_type=jnp.float32,
            precision=jax.lax.Precision.HIGHEST)
        r = r - zqi
        zq = zq + zqi
        loss_sum = loss_sum + jnp.sum(r * r)
    zq_ref[...] = z + (zq - z)
    codes_ref[...] = jnp.stack(codes, axis=1)
    lane = jax.lax.broadcasted_iota(jnp.int32, (1, 1, 8), 2)
    loss_ref[...] = jnp.where(lane == 0, loss_sum, 0.0)


@jax.jit
def kernel(z, codebooks):
    zq, codes, loss_parts = pl.pallas_call(
        _rvq_block_kernel,
        grid=(GRID,),
        in_specs=[
            pl.BlockSpec((BT, D), lambda i: (i, 0)),
            pl.BlockSpec((NUM_CB, K, D), lambda i: (0, 0, 0)),
        ],
        out_specs=[
            pl.BlockSpec((BT, D), lambda i: (i, 0)),
            pl.BlockSpec((BT, NUM_CB), lambda i: (i, 0)),
            pl.BlockSpec((1, 1, 8), lambda i: (i, 0, 0)),
        ],
        out_shape=[
            jax.ShapeDtypeStruct((N, D), jnp.float32),
            jax.ShapeDtypeStruct((N, NUM_CB), jnp.int32),
            jax.ShapeDtypeStruct((GRID, 1, 8), jnp.float32),
        ],
        scratch_shapes=[pltpu.VMEM((NUM_CB, K), jnp.float32)],
    )(z, codebooks)
    loss = jnp.sum(loss_parts) / jnp.float32(N * D)
    return zq, codes, loss, loss
